# baseline (speedup 1.0000x reference)
import functools

import jax
import jax.numpy as jnp
from jax import lax
from jax.experimental import pallas as pl
from jax.experimental.pallas import tpu as pltpu
from jax.experimental.pallas import tpu_sc as plsc


def _segment_sum_sc(attr, idx, num_segments):
  N, F = attr.shape
  info = plsc.get_sparse_core_info()
  NC, NS, L = info.num_cores, info.num_subcores, info.num_lanes
  NW = NC * NS
  rows_per_w = N // NW
  R = 80
  steps = rows_per_w // R
  NBUF = 5
  groups = steps // NBUF

  idx3 = idx.reshape(NW, steps, R)

  mesh = plsc.VectorSubcoreMesh(core_axis_name="c", subcore_axis_name="s")

  @functools.partial(
      pl.kernel,
      mesh=mesh,
      out_type=jax.ShapeDtypeStruct((NC, num_segments, F), jnp.float32),
      scratch_types=[
          pltpu.VMEM((steps, R), jnp.int32),
          [pltpu.VMEM((R, F), jnp.float32) for _ in range(NBUF)],
          pltpu.VMEM((num_segments, F), jnp.float32),
          pltpu.VMEM_SHARED((num_segments, F), jnp.float32),
          pltpu.SemaphoreType.DMA((NBUF,)),
          pltpu.SemaphoreType.DMA((NBUF,)),
      ],
  )
  def k(attr_hbm, idx_hbm, out_hbm, idx_all, rows, zero_v, acc_sh, gsem, ssem):
    cid = lax.axis_index("c")
    sid = lax.axis_index("s")

    @pl.when(sid == 0)
    def _():
      def zrow(i, carry):
        for j in range(F // L):
          zero_v[i, pl.ds(j * L, L)] = jnp.zeros((L,), jnp.float32)
        return carry
      lax.fori_loop(0, num_segments, zrow, 0)
      pltpu.sync_copy(zero_v, acc_sh)

    wid = sid * NC + cid
    base = wid * rows_per_w

    pltpu.sync_copy(idx_hbm.at[wid], idx_all)

    plsc.subcore_barrier()

    LA = NBUF - 2

    def gather(t, b):
      return pltpu.async_copy(
          attr_hbm.at[pl.ds(base + t * R, R)], rows[b], gsem.at[b])

    def wait_gather(t, b):
      pltpu.make_async_copy(
          attr_hbm.at[pl.ds(base + t * R, R)], rows[b], gsem.at[b]).wait()

    def wait_scatter(b):
      pltpu.make_async_copy(
          rows[b], acc_sh.at[idx_all.at[0]], ssem.at[b]).wait()

    for b in range(LA):
      gather(b, b)

    def body(g, carry):
      for b in range(NBUF):
        t = g * NBUF + b
        wait_gather(t, b)
        pltpu.async_copy(rows[b], acc_sh.at[idx_all.at[t]], ssem.at[b],
                         add=True)
        u = t + LA
        bu = (b + LA) % NBUF
        @pl.when(u < steps)
        def _():
          @pl.when(u >= NBUF)
          def _():
            wait_scatter(bu)
          gather(u, bu)
      return carry

    lax.fori_loop(0, groups, body, 0)

    for b in range(NBUF):
      wait_scatter(b)

    plsc.subcore_barrier()

    @pl.when(sid == 0)
    def _():
      pltpu.sync_copy(acc_sh, out_hbm.at[cid])

  return k(attr, idx3)


def kernel(reference, attr, batch_index):
  num_segments = reference.shape[0]
  idx = batch_index.astype(jnp.int32)
  partials = _segment_sum_sc(attr, idx, num_segments)
  return partials[0] + partials[1]


# speedup vs baseline: 1.0689x; 1.0689x over previous
import functools

import jax
import jax.numpy as jnp
from jax import lax
from jax.experimental import pallas as pl
from jax.experimental.pallas import tpu as pltpu
from jax.experimental.pallas import tpu_sc as plsc


def _segment_sum_sc(attr, idx, num_segments):
  N, F = attr.shape
  info = plsc.get_sparse_core_info()
  NC, NS, L = info.num_cores, info.num_subcores, info.num_lanes
  NW = NC * NS
  rows_per_w = N // NW
  R = 80
  steps = rows_per_w // R
  NBUF = 5
  groups = steps // NBUF

  idx3 = idx.reshape(NW, steps, R)

  mesh = plsc.VectorSubcoreMesh(core_axis_name="c", subcore_axis_name="s")

  @functools.partial(
      pl.kernel,
      mesh=mesh,
      out_type=jax.ShapeDtypeStruct((NC, num_segments, F), jnp.float32),
      scratch_types=[
          pltpu.VMEM((steps, R), jnp.int32),
          [pltpu.VMEM((R, F), jnp.float32) for _ in range(NBUF)],
          pltpu.VMEM((num_segments, F), jnp.float32),
          pltpu.VMEM_SHARED((num_segments, F), jnp.float32),
          pltpu.SemaphoreType.DMA((NBUF,)),
          pltpu.SemaphoreType.DMA,
      ],
  )
  def k(attr_hbm, idx_hbm, out_hbm, idx_all, rows, zero_v, acc_sh, gsem, ssem):
    cid = lax.axis_index("c")
    sid = lax.axis_index("s")

    @pl.when(sid == 0)
    def _():
      def zrow(i, carry):
        for j in range(F // L):
          zero_v[i, pl.ds(j * L, L)] = jnp.zeros((L,), jnp.float32)
        return carry
      lax.fori_loop(0, num_segments, zrow, 0)
      pltpu.sync_copy(zero_v, acc_sh)

    wid = sid * NC + cid
    base = wid * rows_per_w

    pltpu.sync_copy(idx_hbm.at[wid], idx_all)

    plsc.subcore_barrier()

    def gather(t, b):
      return pltpu.async_copy(
          attr_hbm.at[pl.ds(base + t * R, R)], rows[b], gsem.at[b])

    for b in range(NBUF):
      gather(b, b)

    def body(g, carry):
      for b in range(NBUF):
        t = g * NBUF + b
        pltpu.make_async_copy(
            attr_hbm.at[pl.ds(base + t * R, R)], rows[b], gsem.at[b]).wait()
        sc = pltpu.async_copy(rows[b], acc_sh.at[idx_all.at[t]], ssem,
                              add=True)
        sc.wait()
        @pl.when(t + NBUF < steps)
        def _():
          gather(t + NBUF, b)
      return carry

    lax.fori_loop(0, groups, body, 0)

    plsc.subcore_barrier()

    @pl.when(sid == 0)
    def _():
      pltpu.sync_copy(acc_sh, out_hbm.at[cid])

  return k(attr, idx3)


def kernel(reference, attr, batch_index):
  num_segments = reference.shape[0]
  idx = batch_index.astype(jnp.int32)
  partials = _segment_sum_sc(attr, idx, num_segments)
  return partials[0] + partials[1]


# speedup vs baseline: 1.6310x; 1.5259x over previous
import functools

import jax
import jax.numpy as jnp
from jax import lax
from jax.experimental import pallas as pl
from jax.experimental.pallas import tpu as pltpu
from jax.experimental.pallas import tpu_sc as plsc


def _segment_sum_sc(attr, idx, num_segments):
  N, F = attr.shape
  info = plsc.get_sparse_core_info()
  NC, NS, L = info.num_cores, info.num_subcores, info.num_lanes
  NW = NC * NS
  NF = F // L
  rows_per_w = N // NW
  R = 80
  steps = rows_per_w // R
  NBUF = 4
  NG = R // L

  idx3 = idx.reshape(NW, steps, R)

  mesh = plsc.VectorSubcoreMesh(core_axis_name="c", subcore_axis_name="s")

  @functools.partial(
      pl.kernel,
      mesh=mesh,
      out_type=jax.ShapeDtypeStruct((NC, num_segments, F), jnp.float32),
      scratch_types=[
          pltpu.VMEM((steps, R), jnp.int32),
          pltpu.VMEM((NBUF, R, F), jnp.float32),
          pltpu.VMEM((num_segments, F), jnp.float32),
          pltpu.VMEM((num_segments,), jnp.int32),
          pltpu.VMEM_SHARED((num_segments, F), jnp.float32),
          pltpu.SemaphoreType.DMA((NBUF,)),
      ],
  )
  def k(attr_hbm, idx_hbm, out_hbm, idx_all, rows, acc_v, iota_v, acc_sh,
        gsem):
    cid = lax.axis_index("c")
    sid = lax.axis_index("s")
    wid = sid * NC + cid
    base = wid * rows_per_w

    def zrow(i, carry):
      for j in range(NF):
        acc_v[i, pl.ds(j * L, L)] = jnp.zeros((L,), jnp.float32)
      return carry
    lax.fori_loop(0, num_segments, zrow, 0)

    @pl.when(sid == 0)
    def _():
      pltpu.sync_copy(acc_v, acc_sh)

    for kk in range(num_segments // L):
      iota_v[pl.ds(kk * L, L)] = lax.iota(jnp.int32, L) + (kk * L)

    pltpu.sync_copy(idx_hbm.at[wid], idx_all)

    plsc.subcore_barrier()

    def gather(t, bi):
      return pltpu.async_copy(
          attr_hbm.at[pl.ds(base + t * R, R)], rows.at[bi], gsem.at[bi])

    def wait_gather(t, bi):
      pltpu.make_async_copy(
          attr_hbm.at[pl.ds(base + t * R, R)], rows.at[bi], gsem.at[bi]
      ).wait()

    for b in range(NBUF):
      gather(b, b)

    def body(t, carry):
      bi = t & (NBUF - 1)
      wait_gather(t, bi)
      for gi in range(NG):
        a = gi * L
        ids = idx_all[t, pl.ds(a, L)]
        s0 = ids[0]
        s15 = ids[L - 1]

        @pl.when(s0 == s15)
        def _():
          acc = tuple(rows[bi, a, pl.ds(j * L, L)] for j in range(NF))
          for r in range(1, L):
            acc = tuple(acc[j] + rows[bi, a + r, pl.ds(j * L, L)]
                        for j in range(NF))
          for j in range(NF):
            plsc.addupdate(acc_v.at[s0, pl.ds(j * L, L)], acc[j])

        @pl.when(s0 != s15)
        def _():
          for r in range(L):
            s_r = ids[r]
            for j in range(NF):
              plsc.addupdate(acc_v.at[s_r, pl.ds(j * L, L)],
                             rows[bi, a + r, pl.ds(j * L, L)])

      @pl.when(t + NBUF < steps)
      def _():
        gather(t + NBUF, bi)
      return carry

    lax.fori_loop(0, steps, body, 0)

    pltpu.sync_copy(acc_v, acc_sh.at[iota_v], add=True)

    plsc.subcore_barrier()

    @pl.when(sid == 0)
    def _():
      pltpu.sync_copy(acc_sh, out_hbm.at[cid])

  return k(attr, idx3)


def kernel(reference, attr, batch_index):
  num_segments = reference.shape[0]
  idx = batch_index.astype(jnp.int32)
  partials = _segment_sum_sc(attr, idx, num_segments)
  return partials[0] + partials[1]
